# Initial kernel scaffold; baseline (speedup 1.0000x reference)
import jax
import jax.numpy as jnp
from jax.experimental import pallas as pl


def kernel(input_ids, embed_W, router_W, w1, w2, proj_W, proj_b, ln_proj_g, ln_proj_b, norm_g, norm_b, Wz, Wc, bz, wconf, bconf, sketch_P, buffer):
    raise NotImplementedError("write your pallas kernel here")


# speedup vs baseline: 1.8181x; 1.8181x over previous
import jax
import jax.numpy as jnp
from jax import lax
from jax.experimental import pallas as pl
from jax.experimental.pallas import tpu as pltpu
from jax.experimental.pallas import tpu_sc as plsc

T = 2048
D = 1024
E = 8
DFF = 1024
V = 16000
SK = 64
NOV_TH = 0.5
REASON_TH = 0.9
MAX_STEPS = 4

_NC, _NS = 2, 16
_NW = _NC * _NS
_BPW = T // _NW


def _gather_body(table_hbm, idx_hbm, out_hbm, idx_v, rows_v, sem):
    wid = lax.axis_index("s") * _NC + lax.axis_index("c")
    base = wid * _BPW
    pltpu.sync_copy(idx_hbm.at[pl.ds(base, _BPW)], idx_v)
    pltpu.async_copy(table_hbm.at[idx_v], rows_v, sem).wait()
    pltpu.sync_copy(rows_v, out_hbm.at[pl.ds(base, _BPW)])


def _sc_gather(table, ids):
    mesh = plsc.VectorSubcoreMesh(core_axis_name="c", subcore_axis_name="s")
    f = pl.kernel(
        _gather_body,
        mesh=mesh,
        out_type=jax.ShapeDtypeStruct((T, D), jnp.float32),
        scratch_types=[
            pltpu.VMEM((_BPW,), jnp.int32),
            pltpu.VMEM((_BPW, D), jnp.float32),
            pltpu.SemaphoreType.DMA,
        ],
    )
    return f(table, ids)


def _moe_body(x_ref, rw_ref, sp_ref, buf_ref, ng_ref, nb_ref, w1_ref, w2_ref,
              stats_ref, zsum_ref, deep_ref, shallow_ref, moe_ref, msum_ref,
              gate_ref):
    e = pl.program_id(0)

    @pl.when(e == 0)
    def _router_part():
        x = x_ref[...]
        mu_x = jnp.mean(x, axis=-1, keepdims=True)
        vv_x = jnp.mean((x - mu_x) ** 2, axis=-1, keepdims=True)
        shallow_ref[...] = ((x - mu_x) / jnp.sqrt(vv_x + 1e-5) * ng_ref[...]
                           + nb_ref[...]).astype(jnp.bfloat16)
        rl = jnp.dot(x, rw_ref[...], preferred_element_type=jnp.float32)
        m = jnp.max(rl, axis=-1, keepdims=True)
        ex = jnp.exp(rl - m)
        se = jnp.sum(ex, axis=-1, keepdims=True)
        probs = ex / se
        lse = m[:, 0] + jnp.log(se[:, 0])
        ii = lax.broadcasted_iota(jnp.int32, (T, E), 1)
        v1 = jnp.max(probs, axis=-1)
        a1 = jnp.min(jnp.where(probs == v1[:, None], ii, E), axis=-1)
        m1 = ii == a1[:, None]
        probs2 = jnp.where(m1, -jnp.inf, probs)
        v2 = jnp.max(probs2, axis=-1)
        a2 = jnp.min(jnp.where(probs2 == v2[:, None], ii, E), axis=-1)
        m2 = ii == a2[:, None]
        ws = v1 + v2 + 1e-9
        gate = (jnp.where(m1, (v1 / ws)[:, None], 0.0)
                + jnp.where(m2, (v2 / ws)[:, None], 0.0))
        gate_ref[...] = gate
        zero_row = jnp.zeros((E, 128), jnp.float32)
        stats_ref[0:1, :] = jnp.zeros((1, 128), jnp.float32) + jnp.sum(lse * lse)
        stats_ref[1:1 + E, :] = zero_row + jnp.sum(m1.astype(jnp.float32), axis=0)[:, None]
        stats_ref[1 + E:1 + 2 * E, :] = zero_row + jnp.sum(probs, axis=0)[:, None]
        zsum = jnp.sum(x, axis=0, keepdims=True)
        zsum_ref[...] = zsum

        z = zsum / float(T)
        s = jnp.dot(z, sp_ref[...], preferred_element_type=jnp.float32)
        s = s / (jnp.sqrt(jnp.sum(s * s)) + 1e-8)
        b = buf_ref[...]
        bn = b / (jnp.sqrt(jnp.sum(b * b, axis=-1, keepdims=True)) + 1e-8)
        scores = lax.dot_general(s, bn, (((1,), (1,)), ((), ())),
                                 preferred_element_type=jnp.float32)
        nov = 1.0 - jnp.max(scores)
        flag = jnp.where(nov > NOV_TH, 1.0, 0.0)
        deep_ref[...] = jnp.zeros((1, 128), jnp.float32) + flag

    xb = x_ref[...].astype(jnp.bfloat16)
    h = jnp.maximum(
        jnp.dot(xb, w1_ref[0].astype(jnp.bfloat16),
                preferred_element_type=jnp.float32), 0.0)
    y = jnp.dot(h.astype(jnp.bfloat16), w2_ref[0].astype(jnp.bfloat16),
                preferred_element_type=jnp.float32)
    ci = lax.broadcasted_iota(jnp.int32, (T, E), 1)
    g = jnp.sum(jnp.where(ci == e, gate_ref[...], 0.0), axis=1, keepdims=True)
    contrib = y * g

    @pl.when(e == 0)
    def _():
        moe_ref[...] = contrib

    @pl.when(e > 0)
    def _():
        moe_ref[...] += contrib

    @pl.when(e == E - 1)
    def _():
        msum_ref[...] = jnp.sum(moe_ref[...], axis=0, keepdims=True)


def _moe(x, rw, sketch_P, buffer, norm_g, norm_b, w1, w2):
    return pl.pallas_call(
        _moe_body,
        grid=(E,),
        in_specs=[
            pl.BlockSpec((T, D), lambda e: (0, 0)),
            pl.BlockSpec((D, E), lambda e: (0, 0)),
            pl.BlockSpec((D, SK), lambda e: (0, 0)),
            pl.BlockSpec((1024, SK), lambda e: (0, 0)),
            pl.BlockSpec((1, D), lambda e: (0, 0)),
            pl.BlockSpec((1, D), lambda e: (0, 0)),
            pl.BlockSpec((1, D, DFF), lambda e: (e, 0, 0)),
            pl.BlockSpec((1, DFF, D), lambda e: (e, 0, 0)),
        ],
        out_specs=[
            pl.BlockSpec((1 + 2 * E, 128), lambda e: (0, 0)),
            pl.BlockSpec((1, D), lambda e: (0, 0)),
            pl.BlockSpec((1, 128), lambda e: (0, 0)),
            pl.BlockSpec((T, D), lambda e: (0, 0)),
            pl.BlockSpec((T, D), lambda e: (0, 0)),
            pl.BlockSpec((1, D), lambda e: (0, 0)),
        ],
        out_shape=[
            jax.ShapeDtypeStruct((1 + 2 * E, 128), jnp.float32),
            jax.ShapeDtypeStruct((1, D), jnp.float32),
            jax.ShapeDtypeStruct((1, 128), jnp.float32),
            jax.ShapeDtypeStruct((T, D), jnp.bfloat16),
            jax.ShapeDtypeStruct((T, D), jnp.float32),
            jax.ShapeDtypeStruct((1, D), jnp.float32),
        ],
        scratch_shapes=[pltpu.VMEM((T, E), jnp.float32)],
        interpret=False,
    )(x, rw, sketch_P, buffer, norm_g, norm_b, w1, w2)


_VB = 512


def _logits_body(sh_ref, moe_ref, msum_ref, zsum_ref, pw_ref, pb_ref, lg_ref,
                 lb_ref, wz_ref, wc_ref, bz_ref, wconf_ref, bconf_ref,
                 deep_ref, g_ref, b_ref, e_ref, o_ref, conf_ref, f_scr):
    i = pl.program_id(0)

    @pl.when(i == 0)
    def _():
        mm = (msum_ref[...] / float(T)).astype(jnp.bfloat16)
        zr = jnp.dot(mm, pw_ref[...], preferred_element_type=jnp.float32) + pb_ref[...]
        mu0 = jnp.mean(zr)
        vv0 = jnp.mean((zr - mu0) ** 2)
        z_cur = (zr - mu0) / jnp.sqrt(vv0 + 1e-5) * lg_ref[...] + lb_ref[...]
        ctx = (zsum_ref[...] / float(T)).astype(jnp.bfloat16)
        cw = jnp.dot(ctx, wc_ref[...], preferred_element_type=jnp.float32) + bz_ref[...]
        halted = jnp.zeros((), jnp.bool_)
        conf = jnp.zeros((), jnp.float32)
        for _ in range(MAX_STEPS):
            hstep = jnp.tanh(
                jnp.dot(z_cur.astype(jnp.bfloat16), wz_ref[...],
                        preferred_element_type=jnp.float32) + cw)
            c = jax.nn.sigmoid(jnp.sum(hstep * wconf_ref[...]) + bconf_ref[0, 0])
            z_cur = jnp.where(halted, z_cur, hstep)
            conf = jnp.where(halted, conf, c)
            halted = jnp.logical_or(halted, c > REASON_TH)
        conf_ref[...] = jnp.zeros((1, 128), jnp.float32) + conf

        g = g_ref[...]
        b = b_ref[...]
        t = moe_ref[...] + z_cur
        mu = jnp.mean(t, axis=-1, keepdims=True)
        vv = jnp.mean((t - mu) ** 2, axis=-1, keepdims=True)
        deepo = ((t - mu) / jnp.sqrt(vv + 1e-5) * g + b).astype(jnp.bfloat16)
        flag = deep_ref[0, 0] > 0.0
        f_scr[...] = jnp.where(flag, deepo, sh_ref[...])

    o_ref[...] = lax.dot_general(
        f_scr[...], e_ref[...].astype(jnp.bfloat16),
        (((1,), (1,)), ((), ())),
        preferred_element_type=jnp.float32)


def _logits(shallow, moe, msum, zsum, pw, pb, lg, lb, wz, wc, bz, wconf,
            bconf, deep, norm_g, norm_b, embed_W):
    full = lambda shape: pl.BlockSpec(shape, lambda i: tuple(0 for _ in shape))
    return pl.pallas_call(
        _logits_body,
        grid=(pl.cdiv(V, _VB),),
        in_specs=[
            full((T, D)),
            full((T, D)),
            full((1, D)),
            full((1, D)),
            full((D, D)),
            full((1, D)),
            full((1, D)),
            full((1, D)),
            full((D, D)),
            full((D, D)),
            full((1, D)),
            full((1, D)),
            full((1, 1)),
            full((1, 128)),
            full((1, D)),
            full((1, D)),
            pl.BlockSpec((_VB, D), lambda i: (i, 0)),
        ],
        out_specs=[
            pl.BlockSpec((T, _VB), lambda i: (0, i)),
            full((1, 128)),
        ],
        out_shape=[
            jax.ShapeDtypeStruct((T, V), jnp.float32),
            jax.ShapeDtypeStruct((1, 128), jnp.float32),
        ],
        scratch_shapes=[pltpu.VMEM((T, D), jnp.bfloat16)],
        interpret=False,
    )(shallow, moe, msum, zsum, pw.astype(jnp.bfloat16), pb, lg, lb,
      wz.astype(jnp.bfloat16), wc.astype(jnp.bfloat16), bz, wconf, bconf,
      deep, norm_g, norm_b, embed_W)


def kernel(input_ids, embed_W, router_W, w1, w2, proj_W, proj_b,
           ln_proj_g, ln_proj_b, norm_g, norm_b, Wz, Wc, bz, wconf, bconf,
           sketch_P, buffer):
    ids = input_ids.reshape(T).astype(jnp.int32)
    x = _sc_gather(embed_W, ids)
    stats, zsum, deep, shallow, moe, msum = _moe(
        x, router_W, sketch_P, buffer, norm_g.reshape(1, D),
        norm_b.reshape(1, D), w1, w2)
    logits, confv = _logits(
        shallow, moe, msum, zsum, proj_W, proj_b.reshape(1, D),
        ln_proj_g.reshape(1, D), ln_proj_b.reshape(1, D), Wz, Wc,
        bz.reshape(1, D), wconf.reshape(1, D), bconf.reshape(1, 1),
        deep, norm_g.reshape(1, D), norm_b.reshape(1, D), embed_W)
    logits = logits.reshape(1, T, V)

    dm = deep[0, 0] > 0.0
    lse2 = stats[0, 0]
    f_sum = stats[1:1 + E, 0]
    p_sum = stats[1 + E:1 + 2 * E, 0]
    z_loss = jnp.where(dm, lse2 / T, 0.0)
    aux_loss = jnp.where(dm, E * jnp.sum(f_sum * p_sum) / (T * T), 0.0)
    confidence = jnp.where(deep[0:1, 0] > 0.0, confv[0:1, 0], 0.0)
    return logits, z_loss, aux_loss, confidence
